# baseline (speedup 1.0000x reference)
import jax
import jax.numpy as jnp
import numpy as np
from jax import lax
from jax.experimental import pallas as pl
from jax.experimental.pallas import tpu as pltpu
from jax.experimental.pallas import tpu_sc as plsc

_NB_ATOMS = 256
_SPLINE_SIZE = 101
_SPLINE_RANGE = 4.0
_GRID = np.float32(2.0 * _SPLINE_RANGE / (_SPLINE_SIZE - 1))
_INV_GRID = np.float32(1.0) / _GRID
_HALF = _SPLINE_SIZE // 2

_ROWS = 65536
_NC, _NS, _L = 2, 16, 16
_NW = _NC * _NS
_RW = _ROWS // _NW
_CR = 64
_NCHUNK = _RW // _CR
_TAB = _NB_ATOMS * _SPLINE_SIZE


def _spline_body(x_hbm, coef_hbm, dcoef_hbm, out_hbm, tab_v, dtab_v, x_v,
                 out_v, in_sem0, in_sem1, out_sem0, out_sem1):
    wid = lax.axis_index("s") * _NC + lax.axis_index("c")
    pltpu.sync_copy(coef_hbm, tab_v)
    pltpu.sync_copy(dcoef_hbm, dtab_v)

    lane_iota = lax.iota(jnp.int32, _L)
    inv_grid = _INV_GRID
    fifty = np.float32(_HALF)
    hi99 = np.float32(_SPLINE_SIZE - 2)
    zero = np.float32(0.0)

    in_sems = (in_sem0, in_sem1)
    out_sems = (out_sem0, out_sem1)
    wbase = wid * _RW

    def _in_copy(k, b):
        return pltpu.make_async_copy(
            x_hbm.at[pl.ds(wbase + k * _CR, _CR), :], x_v.at[b], in_sems[b])

    def _out_copy(k, b):
        return pltpu.make_async_copy(
            out_v.at[b], out_hbm.at[pl.ds(wbase + k * _CR, _CR), :],
            out_sems[b])

    _in_copy(0, 0).start()

    @pl.loop(0, _NCHUNK, step=2)
    def _chunk2(k):
        for b in range(2):
            kk = k + b

            @pl.when(kk + 1 < _NCHUNK)
            def _prefetch():
                _in_copy(kk + 1, 1 - b).start()

            _in_copy(kk, b).wait()

            @pl.when(kk >= 2)
            def _drain():
                _out_copy(kk - 2, b).wait()

            @plsc.parallel_loop(0, _CR * _NB_ATOMS, step=_L, unroll=12)
            def _vec(off):
                r = off >> 8
                c = off & (_NB_ATOMS - 1)
                xv = x_v[b, r, pl.ds(c, _L)]
                y50 = xv * inv_grid + fifty
                y2 = jnp.minimum(jnp.maximum(y50, zero), hi99)
                ti = y2.astype(jnp.int32)
                tf = ti.astype(jnp.float32)
                frac = y50 - tf
                col = lane_iota + c
                idx = (ti << 8) + col
                c_lo = plsc.load_gather(tab_v, [idx])
                d = plsc.load_gather(dtab_v, [idx])
                out_v[b, r, pl.ds(c, _L)] = c_lo + frac * d

            _out_copy(kk, b).start()

    _out_copy(_NCHUNK - 2, 0).wait()
    _out_copy(_NCHUNK - 1, 1).wait()


@jax.jit
def _spline_sc(x, tab_t, dtab_t):
    mesh = plsc.VectorSubcoreMesh(core_axis_name="c", subcore_axis_name="s")
    f = pl.kernel(
        _spline_body,
        out_type=jax.ShapeDtypeStruct((_ROWS, _NB_ATOMS), jnp.float32),
        mesh=mesh,
        scratch_types=[
            pltpu.VMEM((_TAB,), jnp.float32),
            pltpu.VMEM(((_SPLINE_SIZE - 1) * _NB_ATOMS,), jnp.float32),
            pltpu.VMEM((2, _CR, _NB_ATOMS), jnp.float32),
            pltpu.VMEM((2, _CR, _NB_ATOMS), jnp.float32),
            pltpu.SemaphoreType.DMA,
            pltpu.SemaphoreType.DMA,
            pltpu.SemaphoreType.DMA,
            pltpu.SemaphoreType.DMA,
        ],
        compiler_params=pltpu.CompilerParams(
            needs_layout_passes=False, use_tc_tiling_on_sc=True),
    )
    return f(x, tab_t, dtab_t)


def kernel(x, coefficients_vect):
    coefs = coefficients_vect.reshape(_NB_ATOMS, _SPLINE_SIZE)
    tab_t = coefs.T.reshape(-1)
    dtab_t = (coefs[:, 1:] - coefs[:, :-1]).T.reshape(-1)
    return _spline_sc(x, tab_t, dtab_t)


# speedup vs baseline: 1.1779x; 1.1779x over previous
import jax
import jax.numpy as jnp
import numpy as np
from jax import lax
from jax.experimental import pallas as pl
from jax.experimental.pallas import tpu as pltpu
from jax.experimental.pallas import tpu_sc as plsc

_NB_ATOMS = 256
_SPLINE_SIZE = 101
_SPLINE_RANGE = 4.0
_GRID = np.float32(2.0 * _SPLINE_RANGE / (_SPLINE_SIZE - 1))
_INV_GRID = np.float32(1.0) / _GRID
_HALF = _SPLINE_SIZE // 2

_ROWS = 65536
_NC, _NS, _L = 2, 16, 16
_NW = _NC * _NS
_RW = _ROWS // _NW
_CR = 64
_NCHUNK = _RW // _CR
_TAB = _NB_ATOMS * _SPLINE_SIZE


def _spline_body(x_hbm, coef_hbm, dcoef_hbm, out_hbm, tab_v, dtab_v, x_v,
                 out_v, in_sem0, in_sem1, out_sem0, out_sem1):
    wid = lax.axis_index("s") * _NC + lax.axis_index("c")
    pltpu.sync_copy(coef_hbm, tab_v)
    pltpu.sync_copy(dcoef_hbm, dtab_v)

    lane_iota = lax.iota(jnp.int32, _L)
    inv_grid = _INV_GRID
    fifty = np.float32(_HALF)
    hi99 = np.float32(_SPLINE_SIZE - 2)
    zero = np.float32(0.0)

    in_sems = (in_sem0, in_sem1)
    out_sems = (out_sem0, out_sem1)
    wbase = wid * _RW

    def _in_copy(k, b):
        return pltpu.make_async_copy(
            x_hbm.at[pl.ds(wbase + k * _CR, _CR), :], x_v.at[b], in_sems[b])

    def _out_copy(k, b):
        return pltpu.make_async_copy(
            out_v.at[b], out_hbm.at[pl.ds(wbase + k * _CR, _CR), :],
            out_sems[b])

    _in_copy(0, 0).start()

    @pl.loop(0, _NCHUNK, step=2)
    def _chunk2(k):
        for b in range(2):
            kk = k + b

            @pl.when(kk + 1 < _NCHUNK)
            def _prefetch():
                _in_copy(kk + 1, 1 - b).start()

            _in_copy(kk, b).wait()

            @pl.when(kk >= 2)
            def _drain():
                _out_copy(kk - 2, b).wait()

            @plsc.parallel_loop(0, _CR * _NB_ATOMS, step=_L, unroll=4)
            def _vec(off):
                r = off >> 8
                c = off & (_NB_ATOMS - 1)
                xv = x_v[b, r, pl.ds(c, _L)]
                y50 = xv * inv_grid + fifty
                y2 = jnp.minimum(jnp.maximum(y50, zero), hi99)
                ti = y2.astype(jnp.int32)
                tf = ti.astype(jnp.float32)
                frac = y50 - tf
                col = lane_iota + c
                idx = (ti << 8) + col
                c_lo = plsc.load_gather(tab_v, [idx])
                d = plsc.load_gather(dtab_v, [idx])
                out_v[b, r, pl.ds(c, _L)] = c_lo + frac * d

            _out_copy(kk, b).start()

    _out_copy(_NCHUNK - 2, 0).wait()
    _out_copy(_NCHUNK - 1, 1).wait()


@jax.jit
def _spline_sc(x, tab_t, dtab_t):
    mesh = plsc.VectorSubcoreMesh(core_axis_name="c", subcore_axis_name="s")
    f = pl.kernel(
        _spline_body,
        out_type=jax.ShapeDtypeStruct((_ROWS, _NB_ATOMS), jnp.float32),
        mesh=mesh,
        scratch_types=[
            pltpu.VMEM((_TAB,), jnp.float32),
            pltpu.VMEM(((_SPLINE_SIZE - 1) * _NB_ATOMS,), jnp.float32),
            pltpu.VMEM((2, _CR, _NB_ATOMS), jnp.float32),
            pltpu.VMEM((2, _CR, _NB_ATOMS), jnp.float32),
            pltpu.SemaphoreType.DMA,
            pltpu.SemaphoreType.DMA,
            pltpu.SemaphoreType.DMA,
            pltpu.SemaphoreType.DMA,
        ],
        compiler_params=pltpu.CompilerParams(
            needs_layout_passes=False, use_tc_tiling_on_sc=True),
    )
    return f(x, tab_t, dtab_t)


def kernel(x, coefficients_vect):
    coefs = coefficients_vect.reshape(_NB_ATOMS, _SPLINE_SIZE)
    tab_t = coefs.T.reshape(-1)
    dtab_t = (coefs[:, 1:] - coefs[:, :-1]).T.reshape(-1)
    return _spline_sc(x, tab_t, dtab_t)


# speedup vs baseline: 1.5399x; 1.3073x over previous
import jax
import jax.numpy as jnp
import numpy as np
from jax import lax
from jax.experimental import pallas as pl
from jax.experimental.pallas import tpu as pltpu
from jax.experimental.pallas import tpu_sc as plsc

_NB_ATOMS = 256
_SPLINE_SIZE = 101
_SPLINE_RANGE = 4.0
_GRID = np.float32(2.0 * _SPLINE_RANGE / (_SPLINE_SIZE - 1))
_INV_GRID = np.float32(1.0) / _GRID
_HALF = _SPLINE_SIZE // 2

_ROWS = 65536
_NC, _NS, _L = 2, 16, 16
_NW = _NC * _NS
_RW = _ROWS // _NW
_CR = 64
_NCHUNK = _RW // _CR
_TAB = _NB_ATOMS * _SPLINE_SIZE


def _spline_body(x_hbm, mid_hbm, dcoef_hbm, out_hbm, mtab_v, dtab_v, x_v,
                 out_v, in_sem0, in_sem1, out_sem0, out_sem1):
    wid = lax.axis_index("s") * _NC + lax.axis_index("c")
    pltpu.sync_copy(mid_hbm, mtab_v)
    pltpu.sync_copy(dcoef_hbm, dtab_v)

    lane_iota = lax.iota(jnp.int32, _L)
    inv_grid = _INV_GRID
    half_lo = np.float32(_HALF - 0.5)
    lo = np.float32(-0.5)
    hi = np.nextafter(np.float32(_SPLINE_SIZE - 2.5), np.float32(100.0),
                      dtype=np.float32)
    magic = np.float32(1.5 * 2.0 ** 23)

    in_sems = (in_sem0, in_sem1)
    out_sems = (out_sem0, out_sem1)
    wbase = wid * _RW

    def _in_copy(k, b):
        return pltpu.make_async_copy(
            x_hbm.at[pl.ds(wbase + k * _CR, _CR), :], x_v.at[b], in_sems[b])

    def _out_copy(k, b):
        return pltpu.make_async_copy(
            out_v.at[b], out_hbm.at[pl.ds(wbase + k * _CR, _CR), :],
            out_sems[b])

    _in_copy(0, 0).start()

    @pl.loop(0, _NCHUNK, step=2)
    def _chunk2(k):
        for b in range(2):
            kk = k + b

            @pl.when(kk + 1 < _NCHUNK)
            def _prefetch():
                _in_copy(kk + 1, 1 - b).start()

            _in_copy(kk, b).wait()

            @pl.when(kk >= 2)
            def _drain():
                _out_copy(kk - 2, b).wait()

            @plsc.parallel_loop(0, _CR * _NB_ATOMS, step=_L, unroll=8)
            def _vec(off):
                r = off >> 8
                c = off & (_NB_ATOMS - 1)
                xv = x_v[b, r, pl.ds(c, _L)]
                y = xv * inv_grid + half_lo
                y2 = jnp.minimum(jnp.maximum(y, lo), hi)
                z = y2 + magic
                tf = z - magic
                fr = y - tf
                zb = plsc.bitcast(z, jnp.int32)
                col = lane_iota + c
                idx = ((zb << 8) & 0x3FF00) + col
                mid = plsc.load_gather(mtab_v, [idx])
                d = plsc.load_gather(dtab_v, [idx])
                out_v[b, r, pl.ds(c, _L)] = mid + fr * d

            _out_copy(kk, b).start()

    _out_copy(_NCHUNK - 2, 0).wait()
    _out_copy(_NCHUNK - 1, 1).wait()


@jax.jit
def _spline_sc(x, mid_t, dtab_t):
    mesh = plsc.VectorSubcoreMesh(core_axis_name="c", subcore_axis_name="s")
    f = pl.kernel(
        _spline_body,
        out_type=jax.ShapeDtypeStruct((_ROWS, _NB_ATOMS), jnp.float32),
        mesh=mesh,
        scratch_types=[
            pltpu.VMEM(((_SPLINE_SIZE - 1) * _NB_ATOMS,), jnp.float32),
            pltpu.VMEM(((_SPLINE_SIZE - 1) * _NB_ATOMS,), jnp.float32),
            pltpu.VMEM((2, _CR, _NB_ATOMS), jnp.float32),
            pltpu.VMEM((2, _CR, _NB_ATOMS), jnp.float32),
            pltpu.SemaphoreType.DMA,
            pltpu.SemaphoreType.DMA,
            pltpu.SemaphoreType.DMA,
            pltpu.SemaphoreType.DMA,
        ],
        compiler_params=pltpu.CompilerParams(
            needs_layout_passes=False, use_tc_tiling_on_sc=True),
    )
    return f(x, mid_t, dtab_t)


def kernel(x, coefficients_vect):
    coefs = coefficients_vect.reshape(_NB_ATOMS, _SPLINE_SIZE)
    dtab_t = (coefs[:, 1:] - coefs[:, :-1]).T.reshape(-1)
    mid_t = (coefs[:, :-1] + np.float32(0.5) *
             (coefs[:, 1:] - coefs[:, :-1])).T.reshape(-1)
    return _spline_sc(x, mid_t, dtab_t)
